# Initial kernel scaffold; baseline (speedup 1.0000x reference)
import jax
import jax.numpy as jnp
from jax.experimental import pallas as pl


def kernel(x, edge_index, edge_attr, params):
    raise NotImplementedError("write your pallas kernel here")


# speedup vs baseline: 2.4888x; 2.4888x over previous
import functools

import jax
import jax.numpy as jnp
from jax import lax
from jax.experimental import pallas as pl
from jax.experimental.pallas import tpu as pltpu
from jax.experimental.pallas import tpu_sc as plsc

N = 10000
E = 320000
D_FEAT = 128
NCOL = 4
H = 32
NCLS = 2

NC, NS = 2, 16
NWORK = NC * NS
CH = 128
E2 = 327680
EW = E2 // NWORK
E2P = E2 // 4
NROW = E2 // CH
NRW = NROW // NWORK
NP2 = 10016
NPC = NP2 // NS

_f32 = jnp.float32


def _sc_mesh():
    return plsc.VectorSubcoreMesh(
        core_axis_name="c", subcore_axis_name="s",
        num_cores=NC, num_subcores=NS)


_SC_PARAMS = dict(
    mesh=_sc_mesh(),
    compiler_params=pltpu.CompilerParams(use_tc_tiling_on_sc=False),
)


def _sc_msg_scatter(h, ea, src2, dst2, zeros_n):
    KI = 10
    BKE = KI * CH
    NBLK = EW // BKE

    @functools.partial(
        pl.kernel,
        out_type=jax.ShapeDtypeStruct((2 * NP2, H), _f32),
        scratch_types=[
            pltpu.VMEM((KI, CH), jnp.int32),
            pltpu.VMEM((KI, CH), jnp.int32),
            pltpu.VMEM((BKE, H), _f32),
            pltpu.VMEM((BKE, H), _f32),
            pltpu.VMEM_SHARED((NP2, H), _f32),
            pltpu.SemaphoreType.DMA,
            pltpu.SemaphoreType.DMA,
        ],
        **_SC_PARAMS,
    )
    def k(h_hbm, ea_hbm, src_hbm, dst_hbm, zero_hbm, out_hbm,
          srcv, dstv, hrows, earows, acc, gsem, ssem):
        c = lax.axis_index("c")
        s = lax.axis_index("s")
        w = s * NC + c
        r0 = s * NPC
        pltpu.sync_copy(zero_hbm.at[pl.ds(r0, NPC)], acc.at[pl.ds(r0, NPC)])
        plsc.subcore_barrier()

        def blk(t, carry):
            base = w * EW + t * BKE
            irow = w * NRW + t * KI
            pltpu.sync_copy(src_hbm.at[pl.ds(irow, KI)], srcv)
            pltpu.sync_copy(dst_hbm.at[pl.ds(irow, KI)], dstv)
            descs = [
                pltpu.async_copy(h_hbm.at[srcv.at[j]],
                                 hrows.at[pl.ds(j * CH, CH)], gsem)
                for j in range(KI)
            ]
            pltpu.sync_copy(ea_hbm.at[pl.ds(base, BKE)], earows)
            for d in descs:
                d.wait()

            @plsc.parallel_loop(0, BKE, 1, unroll=8)
            def _(i):
                for q in range(H // 16):
                    sl = pl.ds(q * 16, 16)
                    earows[i, sl] = jnp.maximum(
                        hrows[i, sl] + earows[i, sl], 0.0)

            sdescs = [
                pltpu.async_copy(earows.at[pl.ds(j * CH, CH)],
                                 acc.at[dstv.at[j]], ssem, add=True)
                for j in range(KI)
            ]
            for d in sdescs:
                d.wait()
            return carry

        lax.fori_loop(0, NBLK, blk, 0)
        plsc.subcore_barrier()
        pltpu.sync_copy(acc.at[pl.ds(r0, NPC)],
                        out_hbm.at[pl.ds(c * NP2 + r0, NPC)])

    return k(h, ea, src2, dst2, zeros_n)


def _sc_pair_add(tab1, tab2, idx1_2, idx2_2, width, ki):
    BKE = ki * CH
    NBLK = EW // BKE

    @functools.partial(
        pl.kernel,
        out_type=jax.ShapeDtypeStruct((E2, width), _f32),
        scratch_types=[
            pltpu.VMEM((ki, CH), jnp.int32),
            pltpu.VMEM((ki, CH), jnp.int32),
            pltpu.VMEM((BKE, width), _f32),
            pltpu.VMEM((BKE, width), _f32),
            pltpu.SemaphoreType.DMA,
            pltpu.SemaphoreType.DMA,
        ],
        **_SC_PARAMS,
    )
    def k(t1, t2, i1, i2, out_hbm, i1v, i2v, rows1, rows2, sem1, sem2):
        w = lax.axis_index("s") * NC + lax.axis_index("c")

        def blk(t, carry):
            base = w * EW + t * BKE
            irow = w * NRW + t * ki
            pltpu.sync_copy(i1.at[pl.ds(irow, ki)], i1v)
            pltpu.sync_copy(i2.at[pl.ds(irow, ki)], i2v)
            d1 = [
                pltpu.async_copy(t1.at[i1v.at[j]],
                                 rows1.at[pl.ds(j * CH, CH)], sem1)
                for j in range(ki)
            ]
            d2 = [
                pltpu.async_copy(t2.at[i2v.at[j]],
                                 rows2.at[pl.ds(j * CH, CH)], sem2)
                for j in range(ki)
            ]
            for d in d1 + d2:
                d.wait()

            @plsc.parallel_loop(0, BKE, 1, unroll=8)
            def _(i):
                for q in range(width // 16):
                    sl = pl.ds(q * 16, 16)
                    rows1[i, sl] = rows1[i, sl] + rows2[i, sl]

            pltpu.sync_copy(rows1, out_hbm.at[pl.ds(base, BKE)])
            return carry

        lax.fori_loop(0, NBLK, blk, 0)

    return k(tab1, tab2, idx1_2, idx2_2)


def _full(shape):
    return pl.BlockSpec(shape, lambda *_: tuple(0 for _ in shape))


def _tc_node_init(x, node_W, node_b):
    def body(x_ref, w_ref, b_ref, o_ref):
        o_ref[...] = jnp.dot(x_ref[...], w_ref[...],
                             preferred_element_type=_f32) + b_ref[...]

    return pl.pallas_call(
        body,
        out_shape=jax.ShapeDtypeStruct((N, H), _f32),
        in_specs=[_full((N, D_FEAT)), _full((D_FEAT, H)), _full((1, H))],
        out_specs=_full((N, H)),
    )(x, node_W, node_b)


def _tc_ea0(eattr_p, K, c0_t, bep=2048):
    grid = (E2P // bep,)

    def body(a_ref, k_ref, c_ref, o_ref):
        o_ref[...] = jnp.dot(a_ref[...], k_ref[...],
                             preferred_element_type=_f32) + c_ref[...]

    return pl.pallas_call(
        body,
        grid=grid,
        out_shape=jax.ShapeDtypeStruct((E2P, 128), _f32),
        in_specs=[
            pl.BlockSpec((bep, 16), lambda i: (i, 0)),
            _full((16, 128)),
            _full((1, 128)),
        ],
        out_specs=pl.BlockSpec((bep, 128), lambda i: (i, 0)),
    )(eattr_p, K, c0_t)


def _tc_node_update(h, agg2, w1, b1, w2f, b2f, ew1a, ew1b):
    def body(h_ref, a_ref, w1_ref, b1_ref, w2_ref, b2_ref, pa_ref, pb_ref,
             ho_ref, p1_ref, p2_ref):
        h0 = h_ref[...]
        z = h0 + a_ref[0:N, :] + a_ref[NP2:NP2 + N, :]
        z = jnp.maximum(jnp.dot(z, w1_ref[...],
                                preferred_element_type=_f32) + b1_ref[...],
                        0.0)
        z = jnp.dot(z, w2_ref[...], preferred_element_type=_f32) + b2_ref[...]
        h2 = (h0 + jnp.maximum(z, 0.0)) * 0.5
        ho_ref[...] = h2
        p1_ref[...] = jnp.dot(h2, pa_ref[...], preferred_element_type=_f32)
        p2_ref[...] = jnp.dot(h2, pb_ref[...], preferred_element_type=_f32)

    nh = jax.ShapeDtypeStruct((N, H), _f32)
    return pl.pallas_call(
        body,
        out_shape=[nh, nh, nh],
        in_specs=[
            _full((N, H)), _full((2 * NP2, H)),
            _full((H, H)), _full((1, H)), _full((H, H)), _full((1, H)),
            _full((H, H)), _full((H, H)),
        ],
        out_specs=[_full((N, H))] * 3,
    )(h, agg2, w1, b1, w2f, b2f, ew1a, ew1b)


def _tc_edge_update(g_p, ea_p, W1c_bd, eb1_t, W2_bd, eb2_t, bep=2048):
    grid = (E2P // bep,)

    def body(g_ref, e_ref, w1_ref, b1_ref, w2_ref, b2_ref, o_ref):
        ea = e_ref[...]
        t = g_ref[...] + jnp.dot(
            ea, w1_ref[...], preferred_element_type=_f32) + b1_ref[...]
        t = jnp.maximum(t, 0.0)
        u = jnp.dot(t, w2_ref[...], preferred_element_type=_f32) + b2_ref[...]
        o_ref[...] = ea + 0.5 * u

    eb = pl.BlockSpec((bep, 128), lambda i: (i, 0))
    return pl.pallas_call(
        body,
        grid=grid,
        out_shape=jax.ShapeDtypeStruct((E2P, 128), _f32),
        in_specs=[eb, eb, _full((128, 128)), _full((1, 128)),
                  _full((128, 128)), _full((1, 128))],
        out_specs=eb,
    )(g_p, ea_p, W1c_bd, eb1_t, W2_bd, eb2_t)


def _tc_q_prep(h, W1a_pad, W1b_pad):
    def body(h_ref, wa_ref, wb_ref, q1_ref, q2_ref):
        rh = jnp.maximum(h_ref[...], 0.0)
        q1_ref[...] = jnp.dot(rh, wa_ref[...], preferred_element_type=_f32)
        q2_ref[...] = jnp.dot(rh, wb_ref[...], preferred_element_type=_f32)

    nq = jax.ShapeDtypeStruct((N, 64), _f32)
    return pl.pallas_call(
        body,
        out_shape=[nq, nq],
        in_specs=[_full((N, H)), _full((H, 64)), _full((H, 64))],
        out_specs=[_full((N, 64))] * 2,
    )(h, W1a_pad, W1b_pad)


def _tc_head(gq, ea, W1c_pad, b1_pad, w2_pad, b2_pad, w3_pad, b3, be=4096):
    grid = (E2 // be,)

    def body(q_ref, e_ref, w1_ref, b1_ref, w2_ref, b2_ref,
             w3_ref, b3_ref, o_ref):
        o1 = q_ref[...] + jnp.dot(
            e_ref[...], w1_ref[...], preferred_element_type=_f32) + b1_ref[...]
        o1 = jnp.maximum(o1, 0.0)
        o2 = jnp.maximum(jnp.dot(o1, w2_ref[...],
                                 preferred_element_type=_f32) + b2_ref[...],
                         0.0)
        o_ref[...] = jnp.dot(o2, w3_ref[...],
                             preferred_element_type=_f32) + b3_ref[...]

    return pl.pallas_call(
        body,
        grid=grid,
        out_shape=jax.ShapeDtypeStruct((E2, NCLS), _f32),
        in_specs=[pl.BlockSpec((be, 64), lambda i: (i, 0)),
                  pl.BlockSpec((be, H), lambda i: (i, 0)),
                  _full((H, 64)), _full((1, 64)), _full((64, H)),
                  _full((1, H)), _full((H, NCLS)), _full((1, NCLS))],
        out_specs=pl.BlockSpec((be, NCLS), lambda i: (i, 0)),
    )(gq, ea, W1c_pad, b1_pad, w2_pad, b2_pad, w3_pad, b3)


def kernel(x, edge_index, edge_attr, params):
    p = params
    src2 = jnp.pad(edge_index[0], (0, E2 - E)).reshape(NROW, CH)
    dst2 = jnp.pad(edge_index[1], (0, E2 - E),
                   constant_values=N).reshape(NROW, CH)
    eattr_p = jnp.pad(edge_attr, ((0, E2 - E), (0, 0))).reshape(E2P, 16)

    eW3 = p['edge_W'].reshape(NCOL, H, H)
    A = jnp.einsum('ch,chj->cj', p['enc_W'], eW3)
    c0 = p['edge_b'] + jnp.einsum('ch,chj->j', p['enc_b'], eW3)
    I4 = jnp.eye(NCOL, dtype=_f32)
    K = jnp.kron(I4, A)
    c0_t = jnp.tile(c0, NCOL)[None]

    h = _tc_node_init(x, p['node_W'], p['node_b'][None])
    ea_p = _tc_ea0(eattr_p, K, c0_t)

    zeros_n = jnp.zeros((NP2, H), _f32)

    for lp in p['layers']:
        s_bn = lp['bn_gamma'] * lax.rsqrt(lp['bn_var'] + 1e-5)
        w2f = lp['w2'] * s_bn[None, :]
        b2f = lp['b2'] * s_bn + lp['bn_beta'] - lp['bn_mean'] * s_bn
        ew1a = lp['ew1'][:H]
        ew1b = lp['ew1'][H:2 * H]
        W1c_bd = jnp.kron(I4, lp['ew1'][2 * H:])
        eb1_t = jnp.tile(lp['eb1'], NCOL)[None]
        W2_bd = jnp.kron(I4, lp['ew2'])
        eb2_t = jnp.tile(lp['eb2'], NCOL)[None]

        agg2 = _sc_msg_scatter(h, ea_p.reshape(E2, H), src2, dst2, zeros_n)
        h, p1, p2 = _tc_node_update(h, agg2, lp['w1'], lp['b1'][None],
                                    w2f, b2f[None], ew1a, ew1b)
        g = _sc_pair_add(p1, p2, src2, dst2, H, 10)
        ea_p = _tc_edge_update(g.reshape(E2P, 128), ea_p, W1c_bd, eb1_t,
                               W2_bd, eb2_t)

    W1a_pad = jnp.pad(p['mlp_w1'][:H], ((0, 0), (0, 14)))
    W1b_pad = jnp.pad(p['mlp_w1'][H:2 * H], ((0, 0), (0, 14)))
    W1c_pad = jnp.pad(p['mlp_w1'][2 * H:], ((0, 0), (0, 14)))
    b1_pad = jnp.pad(p['mlp_b1'], (0, 14))[None]
    w2_pad = jnp.pad(p['mlp_w2'], ((0, 14), (0, 7)))
    b2_pad = jnp.pad(p['mlp_b2'], (0, 7))[None]
    w3_pad = jnp.pad(p['mlp_w3'], ((0, 7), (0, 0)))

    q1, q2 = _tc_q_prep(h, W1a_pad, W1b_pad)
    gq = _sc_pair_add(q1, q2, src2, dst2, 64, 5)
    out = _tc_head(gq, ea_p.reshape(E2, H), W1c_pad, b1_pad,
                   w2_pad, b2_pad, w3_pad, p['mlp_b3'][None])
    return out[:E]


# speedup vs baseline: 2.9380x; 1.1805x over previous
import functools

import jax
import jax.numpy as jnp
from jax import lax
from jax.experimental import pallas as pl
from jax.experimental.pallas import tpu as pltpu
from jax.experimental.pallas import tpu_sc as plsc

N = 10000
E = 320000
D_FEAT = 128
NCOL = 4
H = 32
NCLS = 2

NC, NS = 2, 16
NWORK = NC * NS
CH = 128
KI = 10
BKE = KI * CH
E2 = 327680
EW = E2 // NWORK
NBLK = EW // BKE
E2P = E2 // 4
NROW = E2 // CH
NRW = NROW // NWORK
NP2 = 10016
NPC = NP2 // NS

_f32 = jnp.float32


def _sc_mesh():
    return plsc.VectorSubcoreMesh(
        core_axis_name="c", subcore_axis_name="s",
        num_cores=NC, num_subcores=NS)


_SC_PARAMS = dict(
    mesh=_sc_mesh(),
    compiler_params=pltpu.CompilerParams(use_tc_tiling_on_sc=False),
)


def _sc_gather(table, idx2):
    @functools.partial(
        pl.kernel,
        out_type=jax.ShapeDtypeStruct((E2, H), _f32),
        scratch_types=[
            pltpu.VMEM((KI, CH), jnp.int32),
            pltpu.VMEM((BKE, H), _f32),
            pltpu.SemaphoreType.DMA,
        ],
        **_SC_PARAMS,
    )
    def k(tab_hbm, idx_hbm, out_hbm, idxv, rows, sem):
        w = lax.axis_index("s") * NC + lax.axis_index("c")

        def blk(t, carry):
            base = w * EW + t * BKE
            irow = w * NRW + t * KI
            pltpu.sync_copy(idx_hbm.at[pl.ds(irow, KI)], idxv)
            descs = [
                pltpu.async_copy(tab_hbm.at[idxv.at[j]],
                                 rows.at[pl.ds(j * CH, CH)], sem)
                for j in range(KI)
            ]
            for d in descs:
                d.wait()
            pltpu.sync_copy(rows, out_hbm.at[pl.ds(base, BKE)])
            return carry

        lax.fori_loop(0, NBLK, blk, 0)

    return k(table, idx2)


def _sc_pair_gather(tab1, tab2, idx1_2, idx2_2):
    @functools.partial(
        pl.kernel,
        out_type=[jax.ShapeDtypeStruct((E2, H), _f32)] * 2,
        scratch_types=[
            pltpu.VMEM((KI, CH), jnp.int32),
            pltpu.VMEM((KI, CH), jnp.int32),
            pltpu.VMEM((BKE, H), _f32),
            pltpu.VMEM((BKE, H), _f32),
            pltpu.SemaphoreType.DMA,
            pltpu.SemaphoreType.DMA,
        ],
        **_SC_PARAMS,
    )
    def k(t1, t2, i1, i2, o1, o2, i1v, i2v, rows1, rows2, sem1, sem2):
        w = lax.axis_index("s") * NC + lax.axis_index("c")

        def blk(t, carry):
            base = w * EW + t * BKE
            irow = w * NRW + t * KI
            pltpu.sync_copy(i1.at[pl.ds(irow, KI)], i1v)
            pltpu.sync_copy(i2.at[pl.ds(irow, KI)], i2v)
            d1 = [
                pltpu.async_copy(t1.at[i1v.at[j]],
                                 rows1.at[pl.ds(j * CH, CH)], sem1)
                for j in range(KI)
            ]
            d2 = [
                pltpu.async_copy(t2.at[i2v.at[j]],
                                 rows2.at[pl.ds(j * CH, CH)], sem2)
                for j in range(KI)
            ]
            for d in d1:
                d.wait()
            pltpu.sync_copy(rows1, o1.at[pl.ds(base, BKE)])
            for d in d2:
                d.wait()
            pltpu.sync_copy(rows2, o2.at[pl.ds(base, BKE)])
            return carry

        lax.fori_loop(0, NBLK, blk, 0)

    return k(tab1, tab2, idx1_2, idx2_2)


def _sc_scatter_add(m, dst2, zeros_n):
    @functools.partial(
        pl.kernel,
        out_type=jax.ShapeDtypeStruct((2 * NP2, H), _f32),
        scratch_types=[
            pltpu.VMEM((KI, CH), jnp.int32),
            pltpu.VMEM((BKE, H), _f32),
            pltpu.VMEM_SHARED((NP2, H), _f32),
            pltpu.SemaphoreType.DMA,
        ],
        **_SC_PARAMS,
    )
    def k(m_hbm, dst_hbm, zero_hbm, out_hbm, idxv, rows, acc, sem):
        c = lax.axis_index("c")
        s = lax.axis_index("s")
        w = s * NC + c
        r0 = s * NPC
        pltpu.sync_copy(zero_hbm.at[pl.ds(r0, NPC)], acc.at[pl.ds(r0, NPC)])
        plsc.subcore_barrier()

        def blk(t, carry):
            base = w * EW + t * BKE
            irow = w * NRW + t * KI
            pltpu.sync_copy(dst_hbm.at[pl.ds(irow, KI)], idxv)
            pltpu.sync_copy(m_hbm.at[pl.ds(base, BKE)], rows)
            descs = [
                pltpu.async_copy(rows.at[pl.ds(j * CH, CH)],
                                 acc.at[idxv.at[j]], sem, add=True)
                for j in range(KI)
            ]
            for d in descs:
                d.wait()
            return carry

        lax.fori_loop(0, NBLK, blk, 0)
        plsc.subcore_barrier()
        pltpu.sync_copy(acc.at[pl.ds(r0, NPC)],
                        out_hbm.at[pl.ds(c * NP2 + r0, NPC)])

    return k(m, dst2, zeros_n)


def _full(shape):
    return pl.BlockSpec(shape, lambda *_: tuple(0 for _ in shape))


def _ebs(bep, width=128):
    return pl.BlockSpec((bep, width), lambda i: (i, 0))


def _tc_node_init(x, node_W, node_b):
    def body(x_ref, w_ref, b_ref, o_ref):
        o_ref[...] = jnp.dot(x_ref[...], w_ref[...],
                             preferred_element_type=_f32) + b_ref[...]

    return pl.pallas_call(
        body,
        out_shape=jax.ShapeDtypeStruct((N, H), _f32),
        in_specs=[_full((N, D_FEAT)), _full((D_FEAT, H)), _full((1, H))],
        out_specs=_full((N, H)),
    )(x, node_W, node_b)


def _tc_ea0(eattr_p, K, c0_t, bep=2048):
    def body(a_ref, k_ref, c_ref, o_ref):
        o_ref[...] = jnp.dot(a_ref[...], k_ref[...],
                             preferred_element_type=_f32) + c_ref[...]

    return pl.pallas_call(
        body,
        grid=(E2P // bep,),
        out_shape=jax.ShapeDtypeStruct((E2P, 128), _f32),
        in_specs=[pl.BlockSpec((bep, 16), lambda i: (i, 0)),
                  _full((16, 128)), _full((1, 128))],
        out_specs=_ebs(bep),
    )(eattr_p, K, c0_t)


def _tc_msg(hs_p, ea_p, bep=2048):
    def body(h_ref, e_ref, o_ref):
        o_ref[...] = jnp.maximum(h_ref[...] + e_ref[...], 0.0)

    return pl.pallas_call(
        body,
        grid=(E2P // bep,),
        out_shape=jax.ShapeDtypeStruct((E2P, 128), _f32),
        in_specs=[_ebs(bep), _ebs(bep)],
        out_specs=_ebs(bep),
    )(hs_p, ea_p)


def _tc_node_update(h, agg2, w1, b1, w2f, b2f, ew1a, ew1b):
    def body(h_ref, a_ref, w1_ref, b1_ref, w2_ref, b2_ref, pa_ref, pb_ref,
             ho_ref, p1_ref, p2_ref):
        h0 = h_ref[...]
        z = h0 + a_ref[0:N, :] + a_ref[NP2:NP2 + N, :]
        z = jnp.maximum(jnp.dot(z, w1_ref[...],
                                preferred_element_type=_f32) + b1_ref[...],
                        0.0)
        z = jnp.dot(z, w2_ref[...], preferred_element_type=_f32) + b2_ref[...]
        h2 = (h0 + jnp.maximum(z, 0.0)) * 0.5
        ho_ref[...] = h2
        p1_ref[...] = jnp.dot(h2, pa_ref[...], preferred_element_type=_f32)
        p2_ref[...] = jnp.dot(h2, pb_ref[...], preferred_element_type=_f32)

    nh = jax.ShapeDtypeStruct((N, H), _f32)
    return pl.pallas_call(
        body,
        out_shape=[nh, nh, nh],
        in_specs=[
            _full((N, H)), _full((2 * NP2, H)),
            _full((H, H)), _full((1, H)), _full((H, H)), _full((1, H)),
            _full((H, H)), _full((H, H)),
        ],
        out_specs=[_full((N, H))] * 3,
    )(h, agg2, w1, b1, w2f, b2f, ew1a, ew1b)


def _tc_edge_update(g1_p, g2_p, ea_p, W1c_bd, eb1_t, W2_bd, eb2_t, bep=2048):
    def body(g1_ref, g2_ref, e_ref, w1_ref, b1_ref, w2_ref, b2_ref, o_ref):
        ea = e_ref[...]
        t = g1_ref[...] + g2_ref[...] + jnp.dot(
            ea, w1_ref[...], preferred_element_type=_f32) + b1_ref[...]
        t = jnp.maximum(t, 0.0)
        u = jnp.dot(t, w2_ref[...], preferred_element_type=_f32) + b2_ref[...]
        o_ref[...] = ea + 0.5 * u

    return pl.pallas_call(
        body,
        grid=(E2P // bep,),
        out_shape=jax.ShapeDtypeStruct((E2P, 128), _f32),
        in_specs=[_ebs(bep), _ebs(bep), _ebs(bep), _full((128, 128)),
                  _full((1, 128)), _full((128, 128)), _full((1, 128))],
        out_specs=_ebs(bep),
    )(g1_p, g2_p, ea_p, W1c_bd, eb1_t, W2_bd, eb2_t)


def _tc_relu(h):
    def body(h_ref, o_ref):
        o_ref[...] = jnp.maximum(h_ref[...], 0.0)

    return pl.pallas_call(
        body,
        out_shape=jax.ShapeDtypeStruct((N, H), _f32),
        in_specs=[_full((N, H))],
        out_specs=_full((N, H)),
    )(h)


def _tc_head(rg1_p, rg2_p, ea_p, K1A, K1B, K1C, b1_t4, W2k, b2_t4, W3k,
             b3_t4, bep=1024):
    def body(r1_ref, r2_ref, e_ref, ka_ref, kb_ref, kc_ref, b1_ref,
             w2_ref, b2_ref, w3_ref, b3_ref, o_ref):
        o1 = (jnp.dot(r1_ref[...], ka_ref[...], preferred_element_type=_f32)
              + jnp.dot(r2_ref[...], kb_ref[...], preferred_element_type=_f32)
              + jnp.dot(e_ref[...], kc_ref[...], preferred_element_type=_f32)
              + b1_ref[...])
        o1 = jnp.maximum(o1, 0.0)
        o2 = jnp.maximum(jnp.dot(o1, w2_ref[...],
                                 preferred_element_type=_f32) + b2_ref[...],
                         0.0)
        o_ref[...] = jnp.dot(o2, w3_ref[...],
                             preferred_element_type=_f32) + b3_ref[...]

    return pl.pallas_call(
        body,
        grid=(E2P // bep,),
        out_shape=jax.ShapeDtypeStruct((E2P, 8), _f32),
        in_specs=[_ebs(bep), _ebs(bep), _ebs(bep),
                  _full((128, 256)), _full((128, 256)), _full((128, 256)),
                  _full((1, 256)), _full((256, 128)), _full((1, 128)),
                  _full((128, 8)), _full((1, 8))],
        out_specs=pl.BlockSpec((bep, 8), lambda i: (i, 0)),
    )(rg1_p, rg2_p, ea_p, K1A, K1B, K1C, b1_t4, W2k, b2_t4, W3k, b3_t4)


def kernel(x, edge_index, edge_attr, params):
    p = params
    src2 = jnp.pad(edge_index[0], (0, E2 - E)).reshape(NROW, CH)
    dst2 = jnp.pad(edge_index[1], (0, E2 - E),
                   constant_values=N).reshape(NROW, CH)
    eattr_p = jnp.pad(edge_attr, ((0, E2 - E), (0, 0))).reshape(E2P, 16)

    eW3 = p['edge_W'].reshape(NCOL, H, H)
    A = jnp.einsum('ch,chj->cj', p['enc_W'], eW3)
    c0 = p['edge_b'] + jnp.einsum('ch,chj->j', p['enc_b'], eW3)
    I4 = jnp.eye(NCOL, dtype=_f32)
    K = jnp.kron(I4, A)
    c0_t = jnp.tile(c0, NCOL)[None]

    h = _tc_node_init(x, p['node_W'], p['node_b'][None])
    ea_p = _tc_ea0(eattr_p, K, c0_t)

    zeros_n = jnp.zeros((NP2, H), _f32)

    for lp in p['layers']:
        s_bn = lp['bn_gamma'] * lax.rsqrt(lp['bn_var'] + 1e-5)
        w2f = lp['w2'] * s_bn[None, :]
        b2f = lp['b2'] * s_bn + lp['bn_beta'] - lp['bn_mean'] * s_bn
        ew1a = lp['ew1'][:H]
        ew1b = lp['ew1'][H:2 * H]
        W1c_bd = jnp.kron(I4, lp['ew1'][2 * H:])
        eb1_t = jnp.tile(lp['eb1'], NCOL)[None]
        W2_bd = jnp.kron(I4, lp['ew2'])
        eb2_t = jnp.tile(lp['eb2'], NCOL)[None]

        hs = _sc_gather(h, src2)
        m_p = _tc_msg(hs.reshape(E2P, 128), ea_p)
        agg2 = _sc_scatter_add(m_p.reshape(E2, H), dst2, zeros_n)
        h, p1, p2 = _tc_node_update(h, agg2, lp['w1'], lp['b1'][None],
                                    w2f, b2f[None], ew1a, ew1b)
        g1, g2 = _sc_pair_gather(p1, p2, src2, dst2)
        ea_p = _tc_edge_update(g1.reshape(E2P, 128), g2.reshape(E2P, 128),
                               ea_p, W1c_bd, eb1_t, W2_bd, eb2_t)

    W1a_p = jnp.pad(p['mlp_w1'][:H], ((0, 0), (0, 14)))
    W1b_p = jnp.pad(p['mlp_w1'][H:2 * H], ((0, 0), (0, 14)))
    W1c_p = jnp.pad(p['mlp_w1'][2 * H:], ((0, 0), (0, 14)))
    w2_p = jnp.pad(p['mlp_w2'], ((0, 14), (0, 7)))
    w3_p = jnp.pad(p['mlp_w3'], ((0, 7), (0, 0)))
    K1A = jnp.kron(I4, W1a_p)
    K1B = jnp.kron(I4, W1b_p)
    K1C = jnp.kron(I4, W1c_p)
    b1_t4 = jnp.tile(jnp.pad(p['mlp_b1'], (0, 14)), NCOL)[None]
    W2k = jnp.kron(I4, w2_p)
    b2_t4 = jnp.tile(jnp.pad(p['mlp_b2'], (0, 7)), NCOL)[None]
    W3k = jnp.kron(I4, w3_p)
    b3_t4 = jnp.tile(p['mlp_b3'], NCOL)[None]

    rh = _tc_relu(h)
    rg1, rg2 = _sc_pair_gather(rh, rh, src2, dst2)
    out3 = _tc_head(rg1.reshape(E2P, 128), rg2.reshape(E2P, 128), ea_p,
                    K1A, K1B, K1C, b1_t4, W2k, b2_t4, W3k, b3_t4)
    return out3.reshape(E2, NCLS)[:E]


# speedup vs baseline: 2.9689x; 1.0105x over previous
import functools

import jax
import jax.numpy as jnp
from jax import lax
from jax.experimental import pallas as pl
from jax.experimental.pallas import tpu as pltpu
from jax.experimental.pallas import tpu_sc as plsc

N = 10000
E = 320000
D_FEAT = 128
NCOL = 4
H = 32
NCLS = 2

NC, NS = 2, 16
NWORK = NC * NS
CH = 128
KI = 10
BKE = KI * CH
E2 = 327680
EW = E2 // NWORK
NBLK = EW // BKE
E2P = E2 // 4
NROW = E2 // CH
NRW = NROW // NWORK
NP2 = 10016
NPC = NP2 // NS

_f32 = jnp.float32


def _sc_mesh():
    return plsc.VectorSubcoreMesh(
        core_axis_name="c", subcore_axis_name="s",
        num_cores=NC, num_subcores=NS)


_SC_PARAMS = dict(
    mesh=_sc_mesh(),
    compiler_params=pltpu.CompilerParams(use_tc_tiling_on_sc=False),
)


def _sc_gather(table, idx2):
    @functools.partial(
        pl.kernel,
        out_type=jax.ShapeDtypeStruct((E2, H), _f32),
        scratch_types=[
            pltpu.VMEM((KI, CH), jnp.int32),
            pltpu.VMEM((BKE, H), _f32),
            pltpu.SemaphoreType.DMA,
        ],
        **_SC_PARAMS,
    )
    def k(tab_hbm, idx_hbm, out_hbm, idxv, rows, sem):
        w = lax.axis_index("s") * NC + lax.axis_index("c")

        def blk(t, carry):
            base = w * EW + t * BKE
            irow = w * NRW + t * KI
            pltpu.sync_copy(idx_hbm.at[pl.ds(irow, KI)], idxv)
            descs = [
                pltpu.async_copy(tab_hbm.at[idxv.at[j]],
                                 rows.at[pl.ds(j * CH, CH)], sem)
                for j in range(KI)
            ]
            for d in descs:
                d.wait()
            pltpu.sync_copy(rows, out_hbm.at[pl.ds(base, BKE)])
            return carry

        lax.fori_loop(0, NBLK, blk, 0)

    return k(table, idx2)


def _sc_pair_gather(tab1, tab2, idx1_2, idx2_2):
    @functools.partial(
        pl.kernel,
        out_type=[jax.ShapeDtypeStruct((E2, H), _f32)] * 2,
        scratch_types=[
            pltpu.VMEM((KI, CH), jnp.int32),
            pltpu.VMEM((KI, CH), jnp.int32),
            pltpu.VMEM((BKE, H), _f32),
            pltpu.VMEM((BKE, H), _f32),
            pltpu.SemaphoreType.DMA,
            pltpu.SemaphoreType.DMA,
        ],
        **_SC_PARAMS,
    )
    def k(t1, t2, i1, i2, o1, o2, i1v, i2v, rows1, rows2, sem1, sem2):
        w = lax.axis_index("s") * NC + lax.axis_index("c")

        def blk(t, carry):
            base = w * EW + t * BKE
            irow = w * NRW + t * KI
            pltpu.sync_copy(i1.at[pl.ds(irow, KI)], i1v)
            pltpu.sync_copy(i2.at[pl.ds(irow, KI)], i2v)
            d1 = [
                pltpu.async_copy(t1.at[i1v.at[j]],
                                 rows1.at[pl.ds(j * CH, CH)], sem1)
                for j in range(KI)
            ]
            d2 = [
                pltpu.async_copy(t2.at[i2v.at[j]],
                                 rows2.at[pl.ds(j * CH, CH)], sem2)
                for j in range(KI)
            ]
            for d in d1:
                d.wait()
            pltpu.sync_copy(rows1, o1.at[pl.ds(base, BKE)])
            for d in d2:
                d.wait()
            pltpu.sync_copy(rows2, o2.at[pl.ds(base, BKE)])
            return carry

        lax.fori_loop(0, NBLK, blk, 0)

    return k(tab1, tab2, idx1_2, idx2_2)


def _sc_scatter_add(m, dst2, zeros_n):
    @functools.partial(
        pl.kernel,
        out_type=jax.ShapeDtypeStruct((2 * NP2, H), _f32),
        scratch_types=[
            pltpu.VMEM((KI, CH), jnp.int32),
            pltpu.VMEM((BKE, H), _f32),
            pltpu.VMEM_SHARED((NP2, H), _f32),
            pltpu.SemaphoreType.DMA,
        ],
        **_SC_PARAMS,
    )
    def k(m_hbm, dst_hbm, zero_hbm, out_hbm, idxv, rows, acc, sem):
        c = lax.axis_index("c")
        s = lax.axis_index("s")
        w = s * NC + c
        r0 = s * NPC
        pltpu.sync_copy(zero_hbm.at[pl.ds(r0, NPC)], acc.at[pl.ds(r0, NPC)])
        plsc.subcore_barrier()

        def blk(t, carry):
            base = w * EW + t * BKE
            irow = w * NRW + t * KI
            pltpu.sync_copy(dst_hbm.at[pl.ds(irow, KI)], idxv)
            pltpu.sync_copy(m_hbm.at[pl.ds(base, BKE)], rows)
            descs = [
                pltpu.async_copy(rows.at[pl.ds(j * CH, CH)],
                                 acc.at[idxv.at[j]], sem, add=True)
                for j in range(KI)
            ]
            for d in descs:
                d.wait()
            return carry

        lax.fori_loop(0, NBLK, blk, 0)
        plsc.subcore_barrier()
        pltpu.sync_copy(acc.at[pl.ds(r0, NPC)],
                        out_hbm.at[pl.ds(c * NP2 + r0, NPC)])

    return k(m, dst2, zeros_n)


def _full(shape):
    return pl.BlockSpec(shape, lambda *_: tuple(0 for _ in shape))


def _ebs(bep, width=128):
    return pl.BlockSpec((bep, width), lambda i: (i, 0))


def _tc_node_init(x, node_W, node_b):
    def body(x_ref, w_ref, b_ref, o_ref):
        o_ref[...] = jnp.dot(x_ref[...], w_ref[...],
                             preferred_element_type=_f32) + b_ref[...]

    return pl.pallas_call(
        body,
        out_shape=jax.ShapeDtypeStruct((N, H), _f32),
        in_specs=[_full((N, D_FEAT)), _full((D_FEAT, H)), _full((1, H))],
        out_specs=_full((N, H)),
    )(x, node_W, node_b)


def _tc_ea0(eattr_p, encW_t, encb_t, Wk, b_t, bep=1024):
    def body(a_ref, ew_ref, eb_ref, wk_ref, b_ref, o_ref):
        ae = pltpu.repeat(a_ref[...], 32, axis=1)
        e = ae * ew_ref[...] + eb_ref[...]
        o_ref[...] = jnp.dot(e, wk_ref[...],
                             preferred_element_type=_f32) + b_ref[...]

    return pl.pallas_call(
        body,
        grid=(E2P // bep,),
        out_shape=jax.ShapeDtypeStruct((E2P, 128), _f32),
        in_specs=[pl.BlockSpec((bep, 16), lambda i: (i, 0)),
                  _full((1, 512)), _full((1, 512)),
                  _full((512, 128)), _full((1, 128))],
        out_specs=_ebs(bep),
    )(eattr_p, encW_t, encb_t, Wk, b_t)


def _tc_msg(hs_p, ea_p, bep=2048):
    def body(h_ref, e_ref, o_ref):
        o_ref[...] = jnp.maximum(h_ref[...] + e_ref[...], 0.0)

    return pl.pallas_call(
        body,
        grid=(E2P // bep,),
        out_shape=jax.ShapeDtypeStruct((E2P, 128), _f32),
        in_specs=[_ebs(bep), _ebs(bep)],
        out_specs=_ebs(bep),
    )(hs_p, ea_p)


def _tc_node_update(h, agg2, w1, b1, w2, b2, bn_scale, bn_shift, ew1a, ew1b):
    def body(h_ref, a_ref, w1_ref, b1_ref, w2_ref, b2_ref, bs_ref, bb_ref,
             pa_ref, pb_ref, ho_ref, p1_ref, p2_ref):
        h0 = h_ref[...]
        z = h0 + a_ref[0:N, :] + a_ref[NP2:NP2 + N, :]
        z = jnp.maximum(jnp.dot(z, w1_ref[...],
                                preferred_element_type=_f32) + b1_ref[...],
                        0.0)
        z = jnp.dot(z, w2_ref[...], preferred_element_type=_f32) + b2_ref[...]
        z = z * bs_ref[...] + bb_ref[...]
        h2 = (h0 + jnp.maximum(z, 0.0)) * 0.5
        ho_ref[...] = h2
        p1_ref[...] = jnp.dot(h2, pa_ref[...], preferred_element_type=_f32)
        p2_ref[...] = jnp.dot(h2, pb_ref[...], preferred_element_type=_f32)

    nh = jax.ShapeDtypeStruct((N, H), _f32)
    return pl.pallas_call(
        body,
        out_shape=[nh, nh, nh],
        in_specs=[
            _full((N, H)), _full((2 * NP2, H)),
            _full((H, H)), _full((1, H)), _full((H, H)), _full((1, H)),
            _full((1, H)), _full((1, H)),
            _full((H, H)), _full((H, H)),
        ],
        out_specs=[_full((N, H))] * 3,
    )(h, agg2, w1, b1, w2, b2, bn_scale, bn_shift, ew1a, ew1b)


def _tc_edge_update(g1_p, g2_p, ea_p, W1c_bd, eb1_t, W2_bd, eb2_t, bep=2048):
    def body(g1_ref, g2_ref, e_ref, w1_ref, b1_ref, w2_ref, b2_ref, o_ref):
        ea = e_ref[...]
        t = g1_ref[...] + g2_ref[...] + jnp.dot(
            ea, w1_ref[...], preferred_element_type=_f32) + b1_ref[...]
        t = jnp.maximum(t, 0.0)
        u = jnp.dot(t, w2_ref[...], preferred_element_type=_f32) + b2_ref[...]
        o_ref[...] = ea + 0.5 * u

    return pl.pallas_call(
        body,
        grid=(E2P // bep,),
        out_shape=jax.ShapeDtypeStruct((E2P, 128), _f32),
        in_specs=[_ebs(bep), _ebs(bep), _ebs(bep), _full((128, 128)),
                  _full((1, 128)), _full((128, 128)), _full((1, 128))],
        out_specs=_ebs(bep),
    )(g1_p, g2_p, ea_p, W1c_bd, eb1_t, W2_bd, eb2_t)


def _tc_relu(h):
    def body(h_ref, o_ref):
        o_ref[...] = jnp.maximum(h_ref[...], 0.0)

    return pl.pallas_call(
        body,
        out_shape=jax.ShapeDtypeStruct((N, H), _f32),
        in_specs=[_full((N, H))],
        out_specs=_full((N, H)),
    )(h)


def _tc_head(rg1_p, rg2_p, ea_p, K1A, K1B, K1C, b1_t4, W2k, b2_t4, W3k,
             b3_t4, bep=1024):
    def body(r1_ref, r2_ref, e_ref, ka_ref, kb_ref, kc_ref, b1_ref,
             w2_ref, b2_ref, w3_ref, b3_ref, o_ref):
        o1 = (jnp.dot(r1_ref[...], ka_ref[...], preferred_element_type=_f32)
              + jnp.dot(r2_ref[...], kb_ref[...], preferred_element_type=_f32)
              + jnp.dot(e_ref[...], kc_ref[...], preferred_element_type=_f32)
              + b1_ref[...])
        o1 = jnp.maximum(o1, 0.0)
        o2 = jnp.maximum(jnp.dot(o1, w2_ref[...],
                                 preferred_element_type=_f32) + b2_ref[...],
                         0.0)
        o_ref[...] = jnp.dot(o2, w3_ref[...],
                             preferred_element_type=_f32) + b3_ref[...]

    return pl.pallas_call(
        body,
        grid=(E2P // bep,),
        out_shape=jax.ShapeDtypeStruct((E2P, 8), _f32),
        in_specs=[_ebs(bep), _ebs(bep), _ebs(bep),
                  _full((128, 256)), _full((128, 256)), _full((128, 256)),
                  _full((1, 256)), _full((256, 128)), _full((1, 128)),
                  _full((128, 8)), _full((1, 8))],
        out_specs=pl.BlockSpec((bep, 8), lambda i: (i, 0)),
    )(rg1_p, rg2_p, ea_p, K1A, K1B, K1C, b1_t4, W2k, b2_t4, W3k, b3_t4)


def kernel(x, edge_index, edge_attr, params):
    p = params
    src2 = jnp.pad(edge_index[0], (0, E2 - E)).reshape(NROW, CH)
    dst2 = jnp.pad(edge_index[1], (0, E2 - E),
                   constant_values=N).reshape(NROW, CH)
    eattr_p = jnp.pad(edge_attr, ((0, E2 - E), (0, 0))).reshape(E2P, 16)

    I4 = jnp.eye(NCOL, dtype=_f32)
    encW_t = jnp.broadcast_to(p['enc_W'].T[:, None, :],
                              (H, NCOL, NCOL)).reshape(-1)[None]
    encb_t = jnp.broadcast_to(p['enc_b'].T[:, None, :],
                              (H, NCOL, NCOL)).reshape(-1)[None]
    Wk_e = jnp.einsum('km,ctj->tkcmj', I4,
                      p['edge_W'].reshape(NCOL, H, H)).reshape(512, 128)
    eb_t = jnp.tile(p['edge_b'], NCOL)[None]

    h = _tc_node_init(x, p['node_W'], p['node_b'][None])
    ea_p = _tc_ea0(eattr_p, encW_t, encb_t, Wk_e, eb_t)

    zeros_n = jnp.zeros((NP2, H), _f32)

    for lp in p['layers']:
        s_bn = lp['bn_gamma'] / jnp.sqrt(lp['bn_var'] + 1e-5)
        t_bn = lp['bn_beta'] - lp['bn_mean'] * s_bn
        ew1a = lp['ew1'][:H]
        ew1b = lp['ew1'][H:2 * H]
        W1c_bd = jnp.kron(I4, lp['ew1'][2 * H:])
        eb1_t = jnp.tile(lp['eb1'], NCOL)[None]
        W2_bd = jnp.kron(I4, lp['ew2'])
        eb2_t = jnp.tile(lp['eb2'], NCOL)[None]

        hs = _sc_gather(h, src2)
        m_p = _tc_msg(hs.reshape(E2P, 128), ea_p)
        agg2 = _sc_scatter_add(m_p.reshape(E2, H), dst2, zeros_n)
        h, p1, p2 = _tc_node_update(h, agg2, lp['w1'], lp['b1'][None],
                                    lp['w2'], lp['b2'][None],
                                    s_bn[None], t_bn[None], ew1a, ew1b)
        g1, g2 = _sc_pair_gather(p1, p2, src2, dst2)
        ea_p = _tc_edge_update(g1.reshape(E2P, 128), g2.reshape(E2P, 128),
                               ea_p, W1c_bd, eb1_t, W2_bd, eb2_t)

    W1a_p = jnp.pad(p['mlp_w1'][:H], ((0, 0), (0, 14)))
    W1b_p = jnp.pad(p['mlp_w1'][H:2 * H], ((0, 0), (0, 14)))
    W1c_p = jnp.pad(p['mlp_w1'][2 * H:], ((0, 0), (0, 14)))
    w2_p = jnp.pad(p['mlp_w2'], ((0, 14), (0, 7)))
    w3_p = jnp.pad(p['mlp_w3'], ((0, 7), (0, 0)))
    K1A = jnp.kron(I4, W1a_p)
    K1B = jnp.kron(I4, W1b_p)
    K1C = jnp.kron(I4, W1c_p)
    b1_t4 = jnp.tile(jnp.pad(p['mlp_b1'], (0, 14)), NCOL)[None]
    W2k = jnp.kron(I4, w2_p)
    b2_t4 = jnp.tile(jnp.pad(p['mlp_b2'], (0, 7)), NCOL)[None]
    W3k = jnp.kron(I4, w3_p)
    b3_t4 = jnp.tile(p['mlp_b3'], NCOL)[None]

    rh = _tc_relu(h)
    rg1, rg2 = _sc_pair_gather(rh, rh, src2, dst2)
    out3 = _tc_head(rg1.reshape(E2P, 128), rg2.reshape(E2P, 128), ea_p,
                    K1A, K1B, K1C, b1_t4, W2k, b2_t4, W3k, b3_t4)
    return out3.reshape(E2, NCLS)[:E]


# speedup vs baseline: 3.0278x; 1.0198x over previous
import functools

import jax
import jax.numpy as jnp
from jax import lax
from jax.experimental import pallas as pl
from jax.experimental.pallas import tpu as pltpu
from jax.experimental.pallas import tpu_sc as plsc

N = 10000
E = 320000
D_FEAT = 128
NCOL = 4
H = 32
NCLS = 2

NC, NS = 2, 16
NWORK = NC * NS
CH = 128
KI = 10
BKE = KI * CH
E2 = 327680
EW = E2 // NWORK
NBLK = EW // BKE
E2P = E2 // 4
NROW = E2 // CH
NRW = NROW // NWORK
NP2 = 10016
NPC = NP2 // NS

_f32 = jnp.float32


def _sc_mesh():
    return plsc.VectorSubcoreMesh(
        core_axis_name="c", subcore_axis_name="s",
        num_cores=NC, num_subcores=NS)


_SC_PARAMS = dict(
    mesh=_sc_mesh(),
    compiler_params=pltpu.CompilerParams(use_tc_tiling_on_sc=False),
)


def _sc_gather(table, idx2):
    KG = 20
    BKG = KG * CH
    NBG = EW // BKG

    @functools.partial(
        pl.kernel,
        out_type=jax.ShapeDtypeStruct((E2, H), _f32),
        scratch_types=[
            pltpu.VMEM((NRW, CH), jnp.int32),
            pltpu.VMEM((BKG, H), _f32),
            pltpu.SemaphoreType.DMA,
        ],
        **_SC_PARAMS,
    )
    def k(tab_hbm, idx_hbm, out_hbm, idxv, rows, sem):
        w = lax.axis_index("s") * NC + lax.axis_index("c")
        pltpu.sync_copy(idx_hbm.at[pl.ds(w * NRW, NRW)], idxv)

        def blk(t, carry):
            base = w * EW + t * BKG

            def fire(j, carry2):
                pltpu.async_copy(tab_hbm.at[idxv.at[t * KG + j]],
                                 rows.at[pl.ds(j * CH, CH)], sem)
                return carry2

            lax.fori_loop(0, KG, fire, 0)
            d = pltpu.make_async_copy(out_hbm.at[pl.ds(base, CH)],
                                      rows.at[pl.ds(0, CH)], sem)
            for _ in range(KG):
                d.wait()
            pltpu.sync_copy(rows, out_hbm.at[pl.ds(base, BKG)])
            return carry

        lax.fori_loop(0, NBG, blk, 0)

    return k(table, idx2)


def _sc_pair_gather(tab1, tab2, idx1_2, idx2_2):
    @functools.partial(
        pl.kernel,
        out_type=[jax.ShapeDtypeStruct((E2, H), _f32)] * 2,
        scratch_types=[
            pltpu.VMEM((NRW, CH), jnp.int32),
            pltpu.VMEM((NRW, CH), jnp.int32),
            pltpu.VMEM((BKE, H), _f32),
            pltpu.VMEM((BKE, H), _f32),
            pltpu.SemaphoreType.DMA,
            pltpu.SemaphoreType.DMA,
        ],
        **_SC_PARAMS,
    )
    def k(t1, t2, i1, i2, o1, o2, i1v, i2v, rows1, rows2, sem1, sem2):
        w = lax.axis_index("s") * NC + lax.axis_index("c")
        pltpu.sync_copy(i1.at[pl.ds(w * NRW, NRW)], i1v)
        pltpu.sync_copy(i2.at[pl.ds(w * NRW, NRW)], i2v)

        def blk(t, carry):
            base = w * EW + t * BKE

            def fire(j, carry2):
                pltpu.async_copy(t1.at[i1v.at[t * KI + j]],
                                 rows1.at[pl.ds(j * CH, CH)], sem1)
                pltpu.async_copy(t2.at[i2v.at[t * KI + j]],
                                 rows2.at[pl.ds(j * CH, CH)], sem2)
                return carry2

            lax.fori_loop(0, KI, fire, 0)
            d1 = pltpu.make_async_copy(o1.at[pl.ds(base, CH)],
                                       rows1.at[pl.ds(0, CH)], sem1)
            d2 = pltpu.make_async_copy(o2.at[pl.ds(base, CH)],
                                       rows2.at[pl.ds(0, CH)], sem2)
            for _ in range(KI):
                d1.wait()
            pltpu.sync_copy(rows1, o1.at[pl.ds(base, BKE)])
            for _ in range(KI):
                d2.wait()
            pltpu.sync_copy(rows2, o2.at[pl.ds(base, BKE)])
            return carry

        lax.fori_loop(0, NBLK, blk, 0)

    return k(tab1, tab2, idx1_2, idx2_2)


def _sc_scatter_add(m, dst2, zeros_n):
    KG = 20
    BKG = KG * CH
    NBG = EW // BKG

    @functools.partial(
        pl.kernel,
        out_type=jax.ShapeDtypeStruct((2 * NP2, H), _f32),
        scratch_types=[
            pltpu.VMEM((NRW, CH), jnp.int32),
            pltpu.VMEM((BKG, H), _f32),
            pltpu.VMEM_SHARED((NP2, H), _f32),
            pltpu.SemaphoreType.DMA,
        ],
        **_SC_PARAMS,
    )
    def k(m_hbm, dst_hbm, zero_hbm, out_hbm, idxv, rows, acc, sem):
        c = lax.axis_index("c")
        s = lax.axis_index("s")
        w = s * NC + c
        r0 = s * NPC
        pltpu.sync_copy(zero_hbm.at[pl.ds(r0, NPC)], acc.at[pl.ds(r0, NPC)])
        pltpu.sync_copy(dst_hbm.at[pl.ds(w * NRW, NRW)], idxv)
        plsc.subcore_barrier()

        def blk(t, carry):
            base = w * EW + t * BKG
            pltpu.sync_copy(m_hbm.at[pl.ds(base, BKG)], rows)

            def fire(j, carry2):
                pltpu.async_copy(rows.at[pl.ds(j * CH, CH)],
                                 acc.at[idxv.at[t * KG + j]], sem, add=True)
                return carry2

            lax.fori_loop(0, KG, fire, 0)
            d = pltpu.make_async_copy(m_hbm.at[pl.ds(base, CH)],
                                      rows.at[pl.ds(0, CH)], sem)
            for _ in range(KG):
                d.wait()
            return carry

        lax.fori_loop(0, NBG, blk, 0)
        plsc.subcore_barrier()
        pltpu.sync_copy(acc.at[pl.ds(r0, NPC)],
                        out_hbm.at[pl.ds(c * NP2 + r0, NPC)])

    return k(m, dst2, zeros_n)


def _full(shape):
    return pl.BlockSpec(shape, lambda *_: tuple(0 for _ in shape))


def _ebs(bep, width=128):
    return pl.BlockSpec((bep, width), lambda i: (i, 0))


def _tc_node_init(x, node_W, node_b):
    def body(x_ref, w_ref, b_ref, o_ref):
        o_ref[...] = jnp.dot(x_ref[...], w_ref[...],
                             preferred_element_type=_f32) + b_ref[...]

    return pl.pallas_call(
        body,
        out_shape=jax.ShapeDtypeStruct((N, H), _f32),
        in_specs=[_full((N, D_FEAT)), _full((D_FEAT, H)), _full((1, H))],
        out_specs=_full((N, H)),
    )(x, node_W, node_b)


def _tc_ea0(eattr_p, encW_t, encb_t, Wk, b_t, bep=1024):
    def body(a_ref, ew_ref, eb_ref, wk_ref, b_ref, o_ref):
        ae = pltpu.repeat(a_ref[...], 32, axis=1)
        e = ae * ew_ref[...] + eb_ref[...]
        o_ref[...] = jnp.dot(e, wk_ref[...],
                             preferred_element_type=_f32) + b_ref[...]

    return pl.pallas_call(
        body,
        grid=(E2P // bep,),
        out_shape=jax.ShapeDtypeStruct((E2P, 128), _f32),
        in_specs=[pl.BlockSpec((bep, 16), lambda i: (i, 0)),
                  _full((1, 512)), _full((1, 512)),
                  _full((512, 128)), _full((1, 128))],
        out_specs=_ebs(bep),
    )(eattr_p, encW_t, encb_t, Wk, b_t)


def _tc_msg(hs_p, ea_p, bep=2048):
    def body(h_ref, e_ref, o_ref):
        o_ref[...] = jnp.maximum(h_ref[...] + e_ref[...], 0.0)

    return pl.pallas_call(
        body,
        grid=(E2P // bep,),
        out_shape=jax.ShapeDtypeStruct((E2P, 128), _f32),
        in_specs=[_ebs(bep), _ebs(bep)],
        out_specs=_ebs(bep),
    )(hs_p, ea_p)


def _tc_node_update(h, agg2, w1, b1, w2, b2, bn_scale, bn_shift, ew1a, ew1b):
    def body(h_ref, a_ref, w1_ref, b1_ref, w2_ref, b2_ref, bs_ref, bb_ref,
             pa_ref, pb_ref, ho_ref, p1_ref, p2_ref):
        h0 = h_ref[...]
        z = h0 + a_ref[0:N, :] + a_ref[NP2:NP2 + N, :]
        z = jnp.maximum(jnp.dot(z, w1_ref[...],
                                preferred_element_type=_f32) + b1_ref[...],
                        0.0)
        z = jnp.dot(z, w2_ref[...], preferred_element_type=_f32) + b2_ref[...]
        z = z * bs_ref[...] + bb_ref[...]
        h2 = (h0 + jnp.maximum(z, 0.0)) * 0.5
        ho_ref[...] = h2
        p1_ref[...] = jnp.dot(h2, pa_ref[...], preferred_element_type=_f32)
        p2_ref[...] = jnp.dot(h2, pb_ref[...], preferred_element_type=_f32)

    nh = jax.ShapeDtypeStruct((N, H), _f32)
    return pl.pallas_call(
        body,
        out_shape=[nh, nh, nh],
        in_specs=[
            _full((N, H)), _full((2 * NP2, H)),
            _full((H, H)), _full((1, H)), _full((H, H)), _full((1, H)),
            _full((1, H)), _full((1, H)),
            _full((H, H)), _full((H, H)),
        ],
        out_specs=[_full((N, H))] * 3,
    )(h, agg2, w1, b1, w2, b2, bn_scale, bn_shift, ew1a, ew1b)


def _tc_edge_update(g1_p, g2_p, ea_p, W1c_bd, eb1_t, W2_bd, eb2_t, bep=2048):
    def body(g1_ref, g2_ref, e_ref, w1_ref, b1_ref, w2_ref, b2_ref, o_ref):
        ea = e_ref[...]
        t = g1_ref[...] + g2_ref[...] + jnp.dot(
            ea, w1_ref[...], preferred_element_type=_f32) + b1_ref[...]
        t = jnp.maximum(t, 0.0)
        u = jnp.dot(t, w2_ref[...], preferred_element_type=_f32) + b2_ref[...]
        o_ref[...] = ea + 0.5 * u

    return pl.pallas_call(
        body,
        grid=(E2P // bep,),
        out_shape=jax.ShapeDtypeStruct((E2P, 128), _f32),
        in_specs=[_ebs(bep), _ebs(bep), _ebs(bep), _full((128, 128)),
                  _full((1, 128)), _full((128, 128)), _full((1, 128))],
        out_specs=_ebs(bep),
    )(g1_p, g2_p, ea_p, W1c_bd, eb1_t, W2_bd, eb2_t)


def _tc_relu(h):
    def body(h_ref, o_ref):
        o_ref[...] = jnp.maximum(h_ref[...], 0.0)

    return pl.pallas_call(
        body,
        out_shape=jax.ShapeDtypeStruct((N, H), _f32),
        in_specs=[_full((N, H))],
        out_specs=_full((N, H)),
    )(h)


def _tc_head(rg1_p, rg2_p, ea_p, K1A, K1B, K1C, b1_t4, W2k, b2_t4, W3k,
             b3_t4, bep=1024):
    def body(r1_ref, r2_ref, e_ref, ka_ref, kb_ref, kc_ref, b1_ref,
             w2_ref, b2_ref, w3_ref, b3_ref, o_ref):
        o1 = (jnp.dot(r1_ref[...], ka_ref[...], preferred_element_type=_f32)
              + jnp.dot(r2_ref[...], kb_ref[...], preferred_element_type=_f32)
              + jnp.dot(e_ref[...], kc_ref[...], preferred_element_type=_f32)
              + b1_ref[...])
        o1 = jnp.maximum(o1, 0.0)
        o2 = jnp.maximum(jnp.dot(o1, w2_ref[...],
                                 preferred_element_type=_f32) + b2_ref[...],
                         0.0)
        o_ref[...] = jnp.dot(o2, w3_ref[...],
                             preferred_element_type=_f32) + b3_ref[...]

    return pl.pallas_call(
        body,
        grid=(E2P // bep,),
        out_shape=jax.ShapeDtypeStruct((E2P, 8), _f32),
        in_specs=[_ebs(bep), _ebs(bep), _ebs(bep),
                  _full((128, 256)), _full((128, 256)), _full((128, 256)),
                  _full((1, 256)), _full((256, 128)), _full((1, 128)),
                  _full((128, 8)), _full((1, 8))],
        out_specs=pl.BlockSpec((bep, 8), lambda i: (i, 0)),
    )(rg1_p, rg2_p, ea_p, K1A, K1B, K1C, b1_t4, W2k, b2_t4, W3k, b3_t4)


def kernel(x, edge_index, edge_attr, params):
    p = params
    src2 = jnp.pad(edge_index[0], (0, E2 - E)).reshape(NROW, CH)
    dst2 = jnp.pad(edge_index[1], (0, E2 - E),
                   constant_values=N).reshape(NROW, CH)
    eattr_p = jnp.pad(edge_attr, ((0, E2 - E), (0, 0))).reshape(E2P, 16)

    I4 = jnp.eye(NCOL, dtype=_f32)
    encW_t = jnp.broadcast_to(p['enc_W'].T[:, None, :],
                              (H, NCOL, NCOL)).reshape(-1)[None]
    encb_t = jnp.broadcast_to(p['enc_b'].T[:, None, :],
                              (H, NCOL, NCOL)).reshape(-1)[None]
    Wk_e = jnp.einsum('km,ctj->tkcmj', I4,
                      p['edge_W'].reshape(NCOL, H, H)).reshape(512, 128)
    eb_t = jnp.tile(p['edge_b'], NCOL)[None]

    h = _tc_node_init(x, p['node_W'], p['node_b'][None])
    ea_p = _tc_ea0(eattr_p, encW_t, encb_t, Wk_e, eb_t)

    zeros_n = jnp.zeros((NP2, H), _f32)

    for lp in p['layers']:
        s_bn = lp['bn_gamma'] / jnp.sqrt(lp['bn_var'] + 1e-5)
        t_bn = lp['bn_beta'] - lp['bn_mean'] * s_bn
        ew1a = lp['ew1'][:H]
        ew1b = lp['ew1'][H:2 * H]
        W1c_bd = jnp.kron(I4, lp['ew1'][2 * H:])
        eb1_t = jnp.tile(lp['eb1'], NCOL)[None]
        W2_bd = jnp.kron(I4, lp['ew2'])
        eb2_t = jnp.tile(lp['eb2'], NCOL)[None]

        hs = _sc_gather(h, src2)
        m_p = _tc_msg(hs.reshape(E2P, 128), ea_p)
        agg2 = _sc_scatter_add(m_p.reshape(E2, H), dst2, zeros_n)
        h, p1, p2 = _tc_node_update(h, agg2, lp['w1'], lp['b1'][None],
                                    lp['w2'], lp['b2'][None],
                                    s_bn[None], t_bn[None], ew1a, ew1b)
        g1, g2 = _sc_pair_gather(p1, p2, src2, dst2)
        ea_p = _tc_edge_update(g1.reshape(E2P, 128), g2.reshape(E2P, 128),
                               ea_p, W1c_bd, eb1_t, W2_bd, eb2_t)

    W1a_p = jnp.pad(p['mlp_w1'][:H], ((0, 0), (0, 14)))
    W1b_p = jnp.pad(p['mlp_w1'][H:2 * H], ((0, 0), (0, 14)))
    W1c_p = jnp.pad(p['mlp_w1'][2 * H:], ((0, 0), (0, 14)))
    w2_p = jnp.pad(p['mlp_w2'], ((0, 14), (0, 7)))
    w3_p = jnp.pad(p['mlp_w3'], ((0, 7), (0, 0)))
    K1A = jnp.kron(I4, W1a_p)
    K1B = jnp.kron(I4, W1b_p)
    K1C = jnp.kron(I4, W1c_p)
    b1_t4 = jnp.tile(jnp.pad(p['mlp_b1'], (0, 14)), NCOL)[None]
    W2k = jnp.kron(I4, w2_p)
    b2_t4 = jnp.tile(jnp.pad(p['mlp_b2'], (0, 7)), NCOL)[None]
    W3k = jnp.kron(I4, w3_p)
    b3_t4 = jnp.tile(p['mlp_b3'], NCOL)[None]

    rh = _tc_relu(h)
    rg1, rg2 = _sc_pair_gather(rh, rh, src2, dst2)
    out3 = _tc_head(rg1.reshape(E2P, 128), rg2.reshape(E2P, 128), ea_p,
                    K1A, K1B, K1C, b1_t4, W2k, b2_t4, W3k, b3_t4)
    return out3.reshape(E2, NCLS)[:E]
